# baseline (speedup 1.0000x reference)
import functools

import jax
import jax.numpy as jnp
from jax import lax
from jax.experimental import pallas as pl
from jax.experimental.pallas import tpu as pltpu
from jax.experimental.pallas import tpu_sc as plsc

N_EMBD = 4096
HIDDEN = N_EMBD // 4
N_EXPERTS = 64
TOP_K = 8
BM = 1024
NW = 32
LANES = 16


def _mlp_block(x_ref, w1_ref, w2_ref, lg_ref):
    x = x_ref[...]
    h = jax.lax.dot_general(
        x, w1_ref[...], (((1,), (1,)), ((), ())),
        preferred_element_type=jnp.float32)
    h = h * jax.nn.sigmoid(h)
    lg_ref[...] = jax.lax.dot_general(
        w2_ref[...], h, (((1,), (1,)), ((), ())),
        preferred_element_type=jnp.float32)


def _route_sc(M):
    tpw = M // NW
    ngrp = tpw // LANES

    @functools.partial(
        pl.kernel,
        out_type=[jax.ShapeDtypeStruct((N_EXPERTS, M), jnp.float32)] * 3,
        mesh=plsc.VectorSubcoreMesh(core_axis_name="c", subcore_axis_name="s"),
        scratch_types=[
            pltpu.VMEM((N_EXPERTS, tpw), jnp.float32),
            pltpu.VMEM((N_EXPERTS, tpw), jnp.float32),
            pltpu.VMEM((N_EXPERTS, tpw), jnp.float32),
            pltpu.VMEM((N_EXPERTS, tpw), jnp.float32),
        ],
    )
    def route(lg_hbm, ew_hbm, mask_hbm, probs_hbm, lg_v, p_v, mk_v, ew_v):
        wid = lax.axis_index("s") * 2 + lax.axis_index("c")
        base = wid * tpw
        pltpu.sync_copy(lg_hbm.at[:, pl.ds(base, tpw)], lg_v)

        def group_body(g, _):
            lanes = pl.ds(g * LANES, LANES)

            m = lg_v[0, lanes]
            for e in range(1, N_EXPERTS):
                m = jnp.maximum(m, lg_v[e, lanes])
            s = jnp.zeros((LANES,), jnp.float32)
            work = []
            for e in range(N_EXPERTS):
                pe = jnp.exp(lg_v[e, lanes] - m)
                work.append(pe)
                s = s + pe
            inv = jnp.full((LANES,), 1.0, jnp.float32) / s
            for e in range(N_EXPERTS):
                work[e] = work[e] * inv
                p_v[e, lanes] = work[e]
                mk_v[e, lanes] = jnp.zeros((LANES,), jnp.float32)

            def round_body(_, work):
                m = work[0]
                for e in range(1, N_EXPERTS):
                    m = jnp.maximum(m, work[e])
                neg_big = jnp.full((LANES,), -(2 ** 30), jnp.int32)
                win = neg_big
                for e in range(N_EXPERTS):
                    cand = jnp.where(work[e] == m,
                                     jnp.full((LANES,), -e, jnp.int32), neg_big)
                    win = jnp.maximum(win, cand)
                new_work = []
                neg_one = jnp.full((LANES,), -1.0, jnp.float32)
                for e in range(N_EXPERTS):
                    sel = win == jnp.full((LANES,), -e, jnp.int32)
                    mk_v[e, lanes] = mk_v[e, lanes] + jnp.where(
                        sel, jnp.full((LANES,), 1.0, jnp.float32),
                        jnp.full((LANES,), 0.0, jnp.float32))
                    new_work.append(jnp.where(sel, neg_one, work[e]))
                return new_work

            work = lax.fori_loop(0, TOP_K, round_body, work)

            ws = jnp.zeros((LANES,), jnp.float32)
            masked = []
            for e in range(N_EXPERTS):
                me = p_v[e, lanes] * mk_v[e, lanes]
                masked.append(me)
                ws = ws + me
            winv = jnp.full((LANES,), 1.0, jnp.float32) / jnp.maximum(
                ws, jnp.full((LANES,), 1e-8, jnp.float32))
            for e in range(N_EXPERTS):
                ew_v[e, lanes] = masked[e] * winv
            return 0

        lax.fori_loop(0, ngrp, group_body, 0)

        pltpu.sync_copy(ew_v, ew_hbm.at[:, pl.ds(base, tpw)])
        pltpu.sync_copy(mk_v, mask_hbm.at[:, pl.ds(base, tpw)])
        pltpu.sync_copy(p_v, probs_hbm.at[:, pl.ds(base, tpw)])

    return route


def kernel(x, W1, W2):
    B, T, E = x.shape
    M = B * T
    xf = x.reshape(M, E)
    logits = pl.pallas_call(
        _mlp_block,
        grid=(M // BM,),
        in_specs=[
            pl.BlockSpec((BM, E), lambda i: (i, 0)),
            pl.BlockSpec((HIDDEN, E), lambda i: (0, 0)),
            pl.BlockSpec((N_EXPERTS, HIDDEN), lambda i: (0, 0)),
        ],
        out_specs=pl.BlockSpec((N_EXPERTS, BM), lambda i: (0, i)),
        out_shape=jax.ShapeDtypeStruct((N_EXPERTS, M), jnp.float32),
    )(xf, W1, W2)
    ew, mask, probs = _route_sc(M)(logits)
    ew, mask, probs = (o.T.reshape(B, T, N_EXPERTS) for o in (ew, mask, probs))
    return (ew, mask, probs)


# speedup vs baseline: 1.8010x; 1.8010x over previous
import jax
import jax.numpy as jnp
from jax.experimental import pallas as pl

N_EMBD = 4096
HIDDEN = N_EMBD // 4
N_EXPERTS = 64
TOP_K = 8
BM = 1024
RCHUNK = 4


def _router_block(x_ref, w1_ref, w2_ref, ew_ref, mask_ref, probs_ref):
    x = x_ref[...]
    h = jax.lax.dot_general(
        x, w1_ref[...], (((1,), (1,)), ((), ())),
        preferred_element_type=jnp.float32)
    h = h * jax.nn.sigmoid(h)
    logits = jax.lax.dot_general(
        w2_ref[...], h, (((1,), (1,)), ((), ())),
        preferred_element_type=jnp.float32)

    cw = BM // RCHUNK
    inv_idx = jnp.int32(N_EXPERTS - 1) - jax.lax.broadcasted_iota(
        jnp.int32, (N_EXPERTS, cw), 0)
    for c in range(RCHUNK):
        cols = slice(c * cw, (c + 1) * cw)
        lg = logits[:, cols]

        mx = jnp.max(lg, axis=0, keepdims=True)
        e = jnp.exp(lg - mx)
        probs = e / jnp.sum(e, axis=0, keepdims=True)
        probs_ref[:, cols] = probs

        work = lg
        mask = jnp.zeros_like(lg)
        for _ in range(TOP_K):
            m = jnp.max(work, axis=0, keepdims=True)
            is_max = work == m
            cand = jnp.where(is_max, inv_idx, -1)
            win = jnp.max(cand, axis=0, keepdims=True)
            sel = cand == win
            mask = mask + sel.astype(jnp.float32)
            work = jnp.where(sel, -jnp.inf, work)
        mask_ref[:, cols] = mask

        masked = probs * mask
        wsum = jnp.maximum(jnp.sum(masked, axis=0, keepdims=True), 1e-8)
        ew_ref[:, cols] = masked / wsum


def kernel(x, W1, W2):
    B, T, E = x.shape
    M = B * T
    xf = x.reshape(M, E)
    outs = pl.pallas_call(
        _router_block,
        grid=(M // BM,),
        in_specs=[
            pl.BlockSpec((BM, E), lambda i: (i, 0)),
            pl.BlockSpec((HIDDEN, E), lambda i: (0, 0)),
            pl.BlockSpec((N_EXPERTS, HIDDEN), lambda i: (0, 0)),
        ],
        out_specs=[pl.BlockSpec((N_EXPERTS, BM), lambda i: (0, i))] * 3,
        out_shape=[jax.ShapeDtypeStruct((N_EXPERTS, M), jnp.float32)] * 3,
    )(xf, W1, W2)
    ew, mask, probs = (o.T.reshape(B, T, N_EXPERTS) for o in outs)
    return (ew, mask, probs)
